# baseline (speedup 1.0000x reference)
import functools

import jax
import jax.numpy as jnp
from jax import lax
from jax.experimental import pallas as pl
from jax.experimental.pallas import tpu as pltpu
from jax.experimental.pallas import tpu_sc as plsc

B, L, D_MODEL = 1, 2048, 768
H, KV, HD = 12, 12, 64
OUT = H * HD
BLK = 512
NBLK = L // BLK
SCALE = 1.0 / (HD ** 0.5)

NC, NS = 2, 16
NW = NC * NS
RPW = L // NW
_SC_MESH = plsc.VectorSubcoreMesh(
    core_axis_name="c", subcore_axis_name="s",
    num_cores=NC, num_subcores=NS)


def _wid():
    return lax.axis_index("s") * NC + lax.axis_index("c")


@functools.partial(
    pl.kernel,
    out_type=[jax.ShapeDtypeStruct((L,), jnp.int32),
              jax.ShapeDtypeStruct((16,), jnp.int32)],
    mesh=_SC_MESH,
    scratch_types=[pltpu.VMEM((L,), jnp.int32),
                   pltpu.VMEM((L + 16,), jnp.int32),
                   pltpu.VMEM((16,), jnp.int32)],
    compiler_params=pltpu.CompilerParams(needs_layout_passes=False),
)
def _sc_compact(mask_hbm, perm_hbm, nact_hbm, mask_v, perm_v, nact_v):
    @pl.when(_wid() == 0)
    def _():
        pltpu.sync_copy(mask_hbm, mask_v)
        base_iota = lax.iota(jnp.int32, 16)

        def body(c, carry):
            off_a, off_i = carry
            mi = mask_v[pl.ds(c * 16, 16)]
            m = mi > 0
            idx = base_iota + c * 16
            ca = plsc.cumsum(mi)[15]
            ci = 16 - ca
            plsc.store_compressed(perm_v.at[pl.ds(off_a, 16)], idx, mask=m)
            plsc.store_compressed(perm_v.at[pl.ds(L - off_i - ci, 16)], idx,
                                  mask=jnp.logical_not(m))
            return off_a + ca, off_i + ci

        nact, _ = lax.fori_loop(0, L // 16, body,
                                (jnp.int32(0), jnp.int32(0)))
        pltpu.sync_copy(perm_v.at[pl.ds(0, L)], perm_hbm)
        nact_v[...] = jnp.full((16,), nact, jnp.int32)
        pltpu.sync_copy(nact_v, nact_hbm)


@functools.partial(
    pl.kernel,
    out_type=jax.ShapeDtypeStruct((L, OUT), jnp.float32),
    mesh=_SC_MESH,
    scratch_types=[pltpu.VMEM((RPW,), jnp.int32),
                   pltpu.VMEM((RPW, OUT), jnp.float32),
                   pltpu.SemaphoreType.DMA],
)
def _sc_gather(perm_hbm, q_hbm, qc_hbm, idx_v, rows_v, sem):
    base = _wid() * RPW
    pltpu.sync_copy(perm_hbm.at[pl.ds(base, RPW)], idx_v)
    pltpu.async_copy(q_hbm.at[idx_v], rows_v, sem).wait()
    pltpu.sync_copy(rows_v, qc_hbm.at[pl.ds(base, RPW)])


@functools.partial(
    pl.kernel,
    out_type=jax.ShapeDtypeStruct((L, OUT), jnp.float32),
    mesh=_SC_MESH,
    scratch_types=[pltpu.VMEM((RPW,), jnp.int32),
                   pltpu.VMEM((RPW, OUT), jnp.float32),
                   pltpu.SemaphoreType.DMA],
)
def _sc_scatter(perm_hbm, outc_hbm, out_hbm, idx_v, rows_v, sem):
    base = _wid() * RPW
    pltpu.sync_copy(perm_hbm.at[pl.ds(base, RPW)], idx_v)
    pltpu.sync_copy(outc_hbm.at[pl.ds(base, RPW)], rows_v)
    pltpu.async_copy(rows_v, out_hbm.at[idx_v], sem).wait()


def _rotary_cols(x, cos_c, sin_c):
    rl = jnp.concatenate([x[:, 32:], x[:, :32]], axis=1)
    rr = jnp.concatenate([x[:, -32:], x[:, :-32]], axis=1)
    lane = lax.broadcasted_iota(jnp.int32, x.shape, 1)
    first_half = (lane % HD) < (HD // 2)
    roth = jnp.where(first_half, -rl, rr)
    return x * cos_c + roth * sin_c


def _tdot(a, b):
    return lax.dot_general(a, b, (((1,), (1,)), ((), ())),
                           preferred_element_type=jnp.float32)


def _q_body(x_ref, w_ref, cos_ref, sin_ref, q_ref):
    xb = x_ref[0].astype(jnp.bfloat16)
    cos_c = jnp.concatenate([cos_ref[0]] * H, axis=1) * SCALE
    sin_c = jnp.concatenate([sin_ref[0]] * H, axis=1) * SCALE
    q = _tdot(xb, w_ref[...].astype(jnp.bfloat16))
    q_ref[...] = _rotary_cols(q, cos_c, sin_c)


def _kv_body(x_ref, wk_ref, wv_ref, cos_ref, sin_ref, k_ref, ve_ref):
    xb = x_ref[0].astype(jnp.bfloat16)
    cos_c = jnp.concatenate([cos_ref[0]] * H, axis=1)
    sin_c = jnp.concatenate([sin_ref[0]] * H, axis=1)
    k = _tdot(xb, wk_ref[...].astype(jnp.bfloat16))
    k_ref[...] = _rotary_cols(k, cos_c, sin_c).astype(jnp.bfloat16)
    v = _tdot(xb, wv_ref[...].astype(jnp.bfloat16)).astype(jnp.bfloat16)
    ones = jnp.ones((v.shape[0], HD), jnp.bfloat16)
    pieces = []
    for h in range(H):
        pieces.append(v[:, HD * h:HD * (h + 1)])
        pieces.append(ones)
    ve_ref[...] = jnp.concatenate(pieces, axis=1)


def _attn_body(n_ref, qc_ref, k_ref, ve_ref, wo_ref, out_ref, acc_ref):
    i = pl.program_id(0)
    n = n_ref[0]

    @pl.when(i * BLK < n)
    def _compute():
        qb = qc_ref[...].astype(jnp.bfloat16)
        for h in range(H):
            s = _tdot(qb[:, HD * h:HD * (h + 1)],
                      k_ref[:, HD * h:HD * (h + 1)])
            p = jnp.exp(s.astype(jnp.bfloat16))
            o2 = jnp.dot(p, ve_ref[:, 2 * HD * h:2 * HD * (h + 1)],
                         preferred_element_type=jnp.float32)
            acc_ref[:, HD * h:HD * (h + 1)] = o2[:, :HD] / o2[:, HD:HD + 1]
        ob = acc_ref[...].astype(jnp.bfloat16)
        rows = i * BLK + lax.broadcasted_iota(jnp.int32, (BLK, 1), 0)
        rmask = (rows < n).astype(jnp.float32)
        out_ref[...] = _tdot(ob, wo_ref[...].astype(jnp.bfloat16)) * rmask

    @pl.when(i * BLK >= n)
    def _skip():
        out_ref[...] = jnp.zeros((BLK, OUT), jnp.float32)


@jax.jit
def kernel(cos, sin, hidden_states, active_mask, Wqkv, Wo):
    mask_i = active_mask[0].astype(jnp.int32)

    perm, nact = _sc_compact(mask_i)

    q = pl.pallas_call(
        _q_body,
        grid=(NBLK,),
        in_specs=[
            pl.BlockSpec((1, BLK, D_MODEL), lambda i: (0, i, 0)),
            pl.BlockSpec((OUT, D_MODEL), lambda i: (0, 0)),
            pl.BlockSpec((1, BLK, HD), lambda i: (0, i, 0)),
            pl.BlockSpec((1, BLK, HD), lambda i: (0, i, 0)),
        ],
        out_specs=pl.BlockSpec((BLK, OUT), lambda i: (i, 0)),
        out_shape=jax.ShapeDtypeStruct((L, OUT), jnp.float32),
    )(hidden_states, Wqkv, cos, sin)

    qc = _sc_gather(perm, q)

    k, ve = pl.pallas_call(
        _kv_body,
        grid=(NBLK,),
        in_specs=[
            pl.BlockSpec((1, BLK, D_MODEL), lambda i: (0, i, 0)),
            pl.BlockSpec((OUT, D_MODEL), lambda i: (1, 0)),
            pl.BlockSpec((OUT, D_MODEL), lambda i: (2, 0)),
            pl.BlockSpec((1, BLK, HD), lambda i: (0, i, 0)),
            pl.BlockSpec((1, BLK, HD), lambda i: (0, i, 0)),
        ],
        out_specs=[
            pl.BlockSpec((BLK, OUT), lambda i: (i, 0)),
            pl.BlockSpec((BLK, 2 * OUT), lambda i: (i, 0)),
        ],
        out_shape=[
            jax.ShapeDtypeStruct((L, OUT), jnp.bfloat16),
            jax.ShapeDtypeStruct((L, 2 * OUT), jnp.bfloat16),
        ],
    )(hidden_states, Wqkv, Wqkv, cos, sin)

    outc = pl.pallas_call(
        _attn_body,
        grid_spec=pltpu.PrefetchScalarGridSpec(
            num_scalar_prefetch=1,
            grid=(NBLK,),
            in_specs=[
                pl.BlockSpec((BLK, OUT), lambda i, n: (i, 0)),
                pl.BlockSpec((L, OUT), lambda i, n: (0, 0)),
                pl.BlockSpec((L, 2 * OUT), lambda i, n: (0, 0)),
                pl.BlockSpec((OUT, OUT), lambda i, n: (0, 0)),
            ],
            out_specs=pl.BlockSpec((BLK, OUT), lambda i, n: (i, 0)),
            scratch_shapes=[pltpu.VMEM((BLK, OUT), jnp.float32)],
        ),
        out_shape=jax.ShapeDtypeStruct((L, OUT), jnp.float32),
    )(nact, qc, k, ve, Wo)

    out = _sc_scatter(perm, outc)

    return out.reshape(B, L, OUT)


# speedup vs baseline: 1.0656x; 1.0656x over previous
import functools

import jax
import jax.numpy as jnp
from jax import lax
from jax.experimental import pallas as pl
from jax.experimental.pallas import tpu as pltpu
from jax.experimental.pallas import tpu_sc as plsc

B, L, D_MODEL = 1, 2048, 768
H, KV, HD = 12, 12, 64
OUT = H * HD
PBLK = 512
BLK = 256
NBLK = L // BLK
NPBLK = L // PBLK
SCALE = 1.0 / (HD ** 0.5)

NC, NS = 2, 16
NW = NC * NS
RPW = L // NW
_SC_MESH = plsc.VectorSubcoreMesh(
    core_axis_name="c", subcore_axis_name="s",
    num_cores=NC, num_subcores=NS)


def _wid():
    return lax.axis_index("s") * NC + lax.axis_index("c")


@functools.partial(
    pl.kernel,
    out_type=[jax.ShapeDtypeStruct((L,), jnp.int32),
              jax.ShapeDtypeStruct((16,), jnp.int32)],
    mesh=_SC_MESH,
    scratch_types=[pltpu.VMEM((L,), jnp.int32),
                   pltpu.VMEM((L + 16,), jnp.int32),
                   pltpu.VMEM((16,), jnp.int32)],
    compiler_params=pltpu.CompilerParams(needs_layout_passes=False),
)
def _sc_compact(mask_hbm, perm_hbm, nact_hbm, mask_v, perm_v, nact_v):
    @pl.when(_wid() == 0)
    def _():
        pltpu.sync_copy(mask_hbm, mask_v)
        base_iota = lax.iota(jnp.int32, 16)

        def body(c, carry):
            off_a, off_i = carry
            mi = mask_v[pl.ds(c * 16, 16)]
            m = mi > 0
            idx = base_iota + c * 16
            ca = plsc.cumsum(mi)[15]
            ci = 16 - ca
            plsc.store_compressed(perm_v.at[pl.ds(off_a, 16)], idx, mask=m)
            plsc.store_compressed(perm_v.at[pl.ds(L - off_i - ci, 16)], idx,
                                  mask=jnp.logical_not(m))
            return off_a + ca, off_i + ci

        nact, _ = lax.fori_loop(0, L // 16, body,
                                (jnp.int32(0), jnp.int32(0)))
        pltpu.sync_copy(perm_v.at[pl.ds(0, L)], perm_hbm)
        nact_v[...] = jnp.full((16,), nact, jnp.int32)
        pltpu.sync_copy(nact_v, nact_hbm)


@functools.partial(
    pl.kernel,
    out_type=jax.ShapeDtypeStruct((L, OUT), jnp.float32),
    mesh=_SC_MESH,
    scratch_types=[pltpu.VMEM((RPW,), jnp.int32),
                   pltpu.VMEM((RPW, OUT), jnp.float32),
                   pltpu.SemaphoreType.DMA],
)
def _sc_gather(perm_hbm, q_hbm, qc_hbm, idx_v, rows_v, sem):
    base = _wid() * RPW
    pltpu.sync_copy(perm_hbm.at[pl.ds(base, RPW)], idx_v)
    pltpu.async_copy(q_hbm.at[idx_v], rows_v, sem).wait()
    pltpu.sync_copy(rows_v, qc_hbm.at[pl.ds(base, RPW)])


@functools.partial(
    pl.kernel,
    out_type=jax.ShapeDtypeStruct((L, OUT), jnp.float32),
    mesh=_SC_MESH,
    scratch_types=[pltpu.VMEM((RPW,), jnp.int32),
                   pltpu.VMEM((RPW, OUT), jnp.float32),
                   pltpu.SemaphoreType.DMA],
)
def _sc_scatter(perm_hbm, outc_hbm, out_hbm, idx_v, rows_v, sem):
    base = _wid() * RPW
    pltpu.sync_copy(perm_hbm.at[pl.ds(base, RPW)], idx_v)
    pltpu.sync_copy(outc_hbm.at[pl.ds(base, RPW)], rows_v)
    pltpu.async_copy(rows_v, out_hbm.at[idx_v], sem).wait()


def _rotary_cols(x, cos_c, sin_c):
    rl = jnp.concatenate([x[:, 32:], x[:, :32]], axis=1)
    rr = jnp.concatenate([x[:, -32:], x[:, :-32]], axis=1)
    lane = lax.broadcasted_iota(jnp.int32, x.shape, 1)
    first_half = (lane % HD) < (HD // 2)
    roth = jnp.where(first_half, -rl, rr)
    return x * cos_c + roth * sin_c


def _tdot(a, b):
    return lax.dot_general(a, b, (((1,), (1,)), ((), ())),
                           preferred_element_type=jnp.float32)


def _q_body(x_ref, w_ref, cos_ref, sin_ref, q_ref):
    xb = x_ref[0].astype(jnp.bfloat16)
    cos_c = jnp.concatenate([cos_ref[0]] * H, axis=1) * SCALE
    sin_c = jnp.concatenate([sin_ref[0]] * H, axis=1) * SCALE
    q = _tdot(xb, w_ref[...].astype(jnp.bfloat16))
    q_ref[...] = _rotary_cols(q, cos_c, sin_c)


def _kv_body(x_ref, wk_ref, wv_ref, cos_ref, sin_ref, k_ref, ve_ref):
    xb = x_ref[0].astype(jnp.bfloat16)
    cos_c = jnp.concatenate([cos_ref[0]] * H, axis=1)
    sin_c = jnp.concatenate([sin_ref[0]] * H, axis=1)
    k = _tdot(xb, wk_ref[...].astype(jnp.bfloat16))
    k_ref[...] = _rotary_cols(k, cos_c, sin_c).astype(jnp.bfloat16)
    v = _tdot(xb, wv_ref[...].astype(jnp.bfloat16)).astype(jnp.bfloat16)
    ones = jnp.ones((v.shape[0], HD), jnp.bfloat16)
    pieces = []
    for h in range(H):
        pieces.append(v[:, HD * h:HD * (h + 1)])
        pieces.append(ones)
    ve_ref[...] = jnp.concatenate(pieces, axis=1)


def _attn_body(n_ref, qc_ref, k_ref, ve_ref, wo_ref, out_ref, acc_ref):
    i = pl.program_id(0)
    n = n_ref[0]

    @pl.when(i * BLK < n)
    def _compute():
        qb = qc_ref[...].astype(jnp.bfloat16)
        for h in range(H):
            s = _tdot(qb[:, HD * h:HD * (h + 1)],
                      k_ref[:, HD * h:HD * (h + 1)])
            p = jnp.exp(s.astype(jnp.bfloat16))
            o2 = jnp.dot(p, ve_ref[:, 2 * HD * h:2 * HD * (h + 1)],
                         preferred_element_type=jnp.float32)
            acc_ref[:, HD * h:HD * (h + 1)] = o2[:, :HD] / o2[:, HD:HD + 1]
        ob = acc_ref[...].astype(jnp.bfloat16)
        rows = i * BLK + lax.broadcasted_iota(jnp.int32, (BLK, 1), 0)
        rmask = (rows < n).astype(jnp.float32)
        out_ref[...] = _tdot(ob, wo_ref[...].astype(jnp.bfloat16)) * rmask

    @pl.when(i * BLK >= n)
    def _skip():
        out_ref[...] = jnp.zeros((BLK, OUT), jnp.float32)


@jax.jit
def kernel(cos, sin, hidden_states, active_mask, Wqkv, Wo):
    mask_i = active_mask[0].astype(jnp.int32)

    perm, nact = _sc_compact(mask_i)

    q = pl.pallas_call(
        _q_body,
        grid=(NPBLK,),
        in_specs=[
            pl.BlockSpec((1, PBLK, D_MODEL), lambda i: (0, i, 0)),
            pl.BlockSpec((OUT, D_MODEL), lambda i: (0, 0)),
            pl.BlockSpec((1, PBLK, HD), lambda i: (0, i, 0)),
            pl.BlockSpec((1, PBLK, HD), lambda i: (0, i, 0)),
        ],
        out_specs=pl.BlockSpec((PBLK, OUT), lambda i: (i, 0)),
        out_shape=jax.ShapeDtypeStruct((L, OUT), jnp.float32),
    )(hidden_states, Wqkv, cos, sin)

    qc = _sc_gather(perm, q)

    k, ve = pl.pallas_call(
        _kv_body,
        grid=(NPBLK,),
        in_specs=[
            pl.BlockSpec((1, PBLK, D_MODEL), lambda i: (0, i, 0)),
            pl.BlockSpec((OUT, D_MODEL), lambda i: (1, 0)),
            pl.BlockSpec((OUT, D_MODEL), lambda i: (2, 0)),
            pl.BlockSpec((1, PBLK, HD), lambda i: (0, i, 0)),
            pl.BlockSpec((1, PBLK, HD), lambda i: (0, i, 0)),
        ],
        out_specs=[
            pl.BlockSpec((PBLK, OUT), lambda i: (i, 0)),
            pl.BlockSpec((PBLK, 2 * OUT), lambda i: (i, 0)),
        ],
        out_shape=[
            jax.ShapeDtypeStruct((L, OUT), jnp.bfloat16),
            jax.ShapeDtypeStruct((L, 2 * OUT), jnp.bfloat16),
        ],
    )(hidden_states, Wqkv, Wqkv, cos, sin)

    outc = pl.pallas_call(
        _attn_body,
        grid_spec=pltpu.PrefetchScalarGridSpec(
            num_scalar_prefetch=1,
            grid=(NBLK,),
            in_specs=[
                pl.BlockSpec((BLK, OUT), lambda i, n: (i, 0)),
                pl.BlockSpec((L, OUT), lambda i, n: (0, 0)),
                pl.BlockSpec((L, 2 * OUT), lambda i, n: (0, 0)),
                pl.BlockSpec((OUT, OUT), lambda i, n: (0, 0)),
            ],
            out_specs=pl.BlockSpec((BLK, OUT), lambda i, n: (i, 0)),
            scratch_shapes=[pltpu.VMEM((BLK, OUT), jnp.float32)],
        ),
        out_shape=jax.ShapeDtypeStruct((L, OUT), jnp.float32),
    )(nact, qc, k, ve, Wo)

    out = _sc_scatter(perm, outc)

    return out.reshape(B, L, OUT)
